# Initial kernel scaffold; baseline (speedup 1.0000x reference)
import jax
import jax.numpy as jnp
from jax.experimental import pallas as pl


def kernel(positions, node_attrs, sigma, noise_pos, noise_attrs, edge_index, batch, ptr, W_embed, W_noise, W_rad, W_sc, W_lin, W_prod, w_read):
    raise NotImplementedError("write your pallas kernel here")


# speedup vs baseline: 2.1729x; 2.1729x over previous
import functools

import numpy as np
import jax
import jax.numpy as jnp
from jax import lax
from jax.experimental import pallas as pl
from jax.experimental.pallas import tpu as pltpu
from jax.experimental.pallas import tpu_sc as plsc

N = 10000
E = 320000
DA = 16
HID = 128
NRAD = 8
NEMB = 16
G = 64
MAXPOS = 1024

NC = 2
NS = 16
NW = NC * NS
EPT = E // NW
C = 80
NRT = N // NS

BN = 2000
BE = 4000

_f32 = jnp.float32
_i32 = jnp.int32

_LOG_SCALE = float(np.log(1.0 / MAXPOS) / (NEMB // 2))


def _kvec():
    return lax.broadcasted_iota(_i32, (1, NRAD), 1).astype(_f32) + 1.0


def _freqs():
    return jnp.exp(
        lax.broadcasted_iota(_i32, (1, NEMB // 2), 1).astype(_f32) * _LOG_SCALE)


def _giota():
    return lax.broadcasted_iota(_i32, (1, G), 1)

@functools.lru_cache(maxsize=None)
def _mesh():
    return plsc.VectorSubcoreMesh(core_axis_name="c", subcore_axis_name="s",
                                  num_cores=NC, num_subcores=NS)


def _sigmoid(x):
    return 1.0 / (1.0 + jnp.exp(-x))


def _dsilu(x):
    s = _sigmoid(x)
    return s * (1.0 + x * (1.0 - s))


def _zero_buf_rows(buf, nrows):
    z = jnp.zeros((16,), _f32)

    def body(e, carry):
        for j in range(HID // 16):
            buf[e, pl.ds(j * 16, 16)] = z
        return carry

    lax.fori_loop(0, nrows, body, 0)


ZROWS = 640
ZTAIL = N - (NS - 1) * ZROWS


def _zero_acc_stripe(src_buf, acc, s):
    @pl.when(s < NS - 1)
    def _full():
        for q in range(ZROWS // C):
            pltpu.sync_copy(src_buf, acc.at[pl.ds(s * ZROWS + q * C, C)])

    @pl.when(s == NS - 1)
    def _tail():
        for q in range(ZTAIL // C):
            pltpu.sync_copy(src_buf,
                            acc.at[pl.ds((NS - 1) * ZROWS + q * C, C)])


def _dump_acc_stripe(acc, out, s, coff):
    @pl.when(s < NS - 1)
    def _full():
        pltpu.sync_copy(acc.at[pl.ds(s * ZROWS, ZROWS)],
                        out.at[pl.ds(coff + s * ZROWS, ZROWS)])

    @pl.when(s == NS - 1)
    def _tail():
        pltpu.sync_copy(acc.at[pl.ds((NS - 1) * ZROWS, ZTAIL)],
                        out.at[pl.ds(coff + (NS - 1) * ZROWS, ZTAIL)])


@functools.lru_cache(maxsize=None)
def _build_sc_msg_fwd(interpret=False):
    def body(h_hbm, rad_hbm, src_hbm, dst_hbm, out_hbm,
             idx_s, idx_d, rows, rad, acc, sem):
        c = lax.axis_index("c")
        s = lax.axis_index("s")
        base_t = (c * NS + s) * EPT

        _zero_buf_rows(rad, C)
        _zero_acc_stripe(rad, acc, s)
        plsc.subcore_barrier()

        def chunk(g, carry):
            base = base_t + g * C
            pltpu.sync_copy(src_hbm.at[pl.ds(base, C)], idx_s)
            pltpu.sync_copy(dst_hbm.at[pl.ds(base, C)], idx_d)
            pltpu.async_copy(h_hbm.at[idx_s], rows, sem).wait()
            pltpu.sync_copy(rad_hbm.at[pl.ds(base, C)], rad)

            def mrow(e, cy):
                for j in range(HID // 16):
                    sl = pl.ds(j * 16, 16)
                    rows[e, sl] = rows[e, sl] * rad[e, sl]
                return cy

            lax.fori_loop(0, C, mrow, 0)
            pltpu.sync_copy(rows, acc.at[idx_d], add=True)
            return carry

        lax.fori_loop(0, EPT // C, chunk, 0)
        plsc.subcore_barrier()
        _dump_acc_stripe(acc, out_hbm, s, c * N)

    return pl.kernel(
        body,
        out_type=jax.ShapeDtypeStruct((NC * N, HID), _f32),
        mesh=_mesh(),
        scratch_types=[
            pltpu.VMEM((C,), _i32),
            pltpu.VMEM((C,), _i32),
            pltpu.VMEM((C, HID), _f32),
            pltpu.VMEM((C, HID), _f32),
            pltpu.VMEM_SHARED((N, HID), _f32),
            pltpu.SemaphoreType.DMA,
        ],
        interpret=interpret,
    )


@functools.lru_cache(maxsize=None)
def _build_sc_msg_bwd(interpret=False):
    def body(g_hbm, h_hbm, rad_hbm, src_hbm, dst_hbm, ghm_hbm, p_hbm,
             idx_s, idx_d, grows, hrows, rad, acc, sem1, sem2):
        c = lax.axis_index("c")
        s = lax.axis_index("s")
        base_t = (c * NS + s) * EPT

        _zero_buf_rows(rad, C)
        _zero_acc_stripe(rad, acc, s)
        plsc.subcore_barrier()

        def chunk(g, carry):
            base = base_t + g * C
            pltpu.sync_copy(src_hbm.at[pl.ds(base, C)], idx_s)
            pltpu.sync_copy(dst_hbm.at[pl.ds(base, C)], idx_d)
            d1 = pltpu.async_copy(g_hbm.at[idx_d], grows, sem1)
            d2 = pltpu.async_copy(h_hbm.at[idx_s], hrows, sem2)
            pltpu.sync_copy(rad_hbm.at[pl.ds(base, C)], rad)
            d1.wait()
            d2.wait()

            def mrow(e, cy):
                for j in range(HID // 16):
                    sl = pl.ds(j * 16, 16)
                    gv = grows[e, sl]
                    rad[e, sl] = gv * rad[e, sl]
                    hrows[e, sl] = gv * hrows[e, sl]
                return cy

            lax.fori_loop(0, C, mrow, 0)
            pltpu.sync_copy(rad, acc.at[idx_s], add=True)
            pltpu.sync_copy(hrows, p_hbm.at[pl.ds(base, C)])
            return carry

        lax.fori_loop(0, EPT // C, chunk, 0)
        plsc.subcore_barrier()
        _dump_acc_stripe(acc, ghm_hbm, s, c * N)

    return pl.kernel(
        body,
        out_type=(
            jax.ShapeDtypeStruct((NC * N, HID), _f32),
            jax.ShapeDtypeStruct((E, HID), _f32),
        ),
        mesh=_mesh(),
        scratch_types=[
            pltpu.VMEM((C,), _i32),
            pltpu.VMEM((C,), _i32),
            pltpu.VMEM((C, HID), _f32),
            pltpu.VMEM((C, HID), _f32),
            pltpu.VMEM((C, HID), _f32),
            pltpu.VMEM_SHARED((N, HID), _f32),
            pltpu.SemaphoreType.DMA,
            pltpu.SemaphoreType.DMA,
        ],
        interpret=interpret,
    )


@functools.lru_cache(maxsize=None)
def _build_sc_pos_fwd(interpret=False):
    def body(pos_hbm, src_hbm, dst_hbm, ox, oy, oz, or2,
             table, idx_s, idx_d, bx, by, bz, br, sem):
        c = lax.axis_index("c")
        s = lax.axis_index("s")
        base_t = (c * NS + s) * EPT

        pltpu.sync_copy(pos_hbm, table)

        def chunk(g, carry):
            base = base_t + g * C
            pltpu.sync_copy(src_hbm.at[pl.ds(base, C)], idx_s)
            pltpu.sync_copy(dst_hbm.at[pl.ds(base, C)], idx_d)
            for q in range(C // 16):
                sl = pl.ds(q * 16, 16)
                s0 = idx_s[sl] * 4
                d0 = idx_d[sl] * 4
                vx = (plsc.load_gather(table, [d0])
                      - plsc.load_gather(table, [s0]))
                vy = (plsc.load_gather(table, [d0 + 1])
                      - plsc.load_gather(table, [s0 + 1]))
                vz = (plsc.load_gather(table, [d0 + 2])
                      - plsc.load_gather(table, [s0 + 2]))
                bx[sl] = vx
                by[sl] = vy
                bz[sl] = vz
                br[sl] = vx * vx + vy * vy + vz * vz
            pltpu.sync_copy(bx, ox.at[pl.ds(base, C)])
            pltpu.sync_copy(by, oy.at[pl.ds(base, C)])
            pltpu.sync_copy(bz, oz.at[pl.ds(base, C)])
            pltpu.sync_copy(br, or2.at[pl.ds(base, C)])
            return carry

        lax.fori_loop(0, EPT // C, chunk, 0)

    ev = jax.ShapeDtypeStruct((E,), _f32)
    return pl.kernel(
        body,
        out_type=(ev, ev, ev, ev),
        mesh=_mesh(),
        compiler_params=pltpu.CompilerParams(needs_layout_passes=False),
        scratch_types=[
            pltpu.VMEM((N * 4,), _f32),
            pltpu.VMEM((C,), _i32),
            pltpu.VMEM((C,), _i32),
            pltpu.VMEM((C,), _f32),
            pltpu.VMEM((C,), _f32),
            pltpu.VMEM((C,), _f32),
            pltpu.VMEM((C,), _f32),
            pltpu.SemaphoreType.DMA,
        ],
        interpret=interpret,
    )


@functools.lru_cache(maxsize=None)
def _build_sc_pos_bwd(interpret=False):
    def body(vx_hbm, vy_hbm, vz_hbm, a_hbm, src_hbm, dst_hbm,
             gpx_hbm, gpy_hbm, gpz_hbm,
             idx_s, idx_d, bx, by, bz, ba, gxb, gyb, gzb, nxb, nyb, nzb,
             zbuf, dbuf, accx, accy, accz, sem):
        c = lax.axis_index("c")
        s = lax.axis_index("s")
        base_t = (c * NS + s) * EPT

        z = jnp.zeros((16,), _f32)
        for q in range(C // 16):
            zbuf[pl.ds(q * 16, 16)] = z

        @pl.when(s < NS - 1)
        def _zero_full():
            for acc in (accx, accy, accz):
                for q in range(ZROWS // C):
                    pltpu.sync_copy(zbuf, acc.at[pl.ds(s * ZROWS + q * C, C)])

        @pl.when(s == NS - 1)
        def _zero_tail():
            for acc in (accx, accy, accz):
                for q in range(ZTAIL // C):
                    pltpu.sync_copy(
                        zbuf, acc.at[pl.ds((NS - 1) * ZROWS + q * C, C)])

        plsc.subcore_barrier()

        def chunk(g, carry):
            base = base_t + g * C
            pltpu.sync_copy(src_hbm.at[pl.ds(base, C)], idx_s)
            pltpu.sync_copy(dst_hbm.at[pl.ds(base, C)], idx_d)
            pltpu.sync_copy(vx_hbm.at[pl.ds(base, C)], bx)
            pltpu.sync_copy(vy_hbm.at[pl.ds(base, C)], by)
            pltpu.sync_copy(vz_hbm.at[pl.ds(base, C)], bz)
            pltpu.sync_copy(a_hbm.at[pl.ds(base, C)], ba)
            for q in range(C // 16):
                sl = pl.ds(q * 16, 16)
                av = ba[sl]
                gx = av * bx[sl]
                gy = av * by[sl]
                gz = av * bz[sl]
                gxb[sl] = gx
                gyb[sl] = gy
                gzb[sl] = gz
                nxb[sl] = -gx
                nyb[sl] = -gy
                nzb[sl] = -gz
            pltpu.sync_copy(gxb, accx.at[idx_d], add=True)
            pltpu.sync_copy(gyb, accy.at[idx_d], add=True)
            pltpu.sync_copy(gzb, accz.at[idx_d], add=True)
            pltpu.sync_copy(nxb, accx.at[idx_s], add=True)
            pltpu.sync_copy(nyb, accy.at[idx_s], add=True)
            pltpu.sync_copy(nzb, accz.at[idx_s], add=True)
            return carry

        lax.fori_loop(0, EPT // C, chunk, 0)
        plsc.subcore_barrier()

        @pl.when(s < NS - 1)
        def _dump_full():
            for acc, out in ((accx, gpx_hbm), (accy, gpy_hbm), (accz, gpz_hbm)):
                pltpu.sync_copy(acc.at[pl.ds(s * ZROWS, ZROWS)], dbuf)
                pltpu.sync_copy(dbuf, out.at[pl.ds(c * N + s * ZROWS, ZROWS)])

        @pl.when(s == NS - 1)
        def _dump_tail():
            for acc, out in ((accx, gpx_hbm), (accy, gpy_hbm), (accz, gpz_hbm)):
                pltpu.sync_copy(acc.at[pl.ds((NS - 1) * ZROWS, ZTAIL)],
                                dbuf.at[pl.ds(0, ZTAIL)])
                pltpu.sync_copy(dbuf.at[pl.ds(0, ZTAIL)],
                                out.at[pl.ds(c * N + (NS - 1) * ZROWS, ZTAIL)])

    ev2 = jax.ShapeDtypeStruct((NC * N,), _f32)
    return pl.kernel(
        body,
        out_type=(ev2, ev2, ev2),
        mesh=_mesh(),
        scratch_types=[
            pltpu.VMEM((C,), _i32),
            pltpu.VMEM((C,), _i32),
            pltpu.VMEM((C,), _f32),
            pltpu.VMEM((C,), _f32),
            pltpu.VMEM((C,), _f32),
            pltpu.VMEM((C,), _f32),
            pltpu.VMEM((C,), _f32),
            pltpu.VMEM((C,), _f32),
            pltpu.VMEM((C,), _f32),
            pltpu.VMEM((C,), _f32),
            pltpu.VMEM((C,), _f32),
            pltpu.VMEM((C,), _f32),
            pltpu.VMEM((C,), _f32),
            pltpu.VMEM((ZROWS,), _f32),
            pltpu.VMEM_SHARED((N,), _f32),
            pltpu.VMEM_SHARED((N,), _f32),
            pltpu.VMEM_SHARED((N,), _f32),
            pltpu.SemaphoreType.DMA,
        ],
        interpret=interpret,
    )


def _full(shape):
    return pl.BlockSpec(shape, lambda i: tuple(0 for _ in shape))


def _nblk(width):
    return pl.BlockSpec((BN, width), lambda i: (i, 0))


def _eblk(width):
    return pl.BlockSpec((BE, width), lambda i: (i, 0))


def _tc_node0_body(attrs_ref, sig_ref, we_ref, wn_ref, h0_ref, sa_ref):
    sig = sig_ref[...]
    ang = sig * _freqs()
    emb = jnp.concatenate([jnp.cos(ang), jnp.sin(ang)], axis=1)
    pre = jnp.dot(emb, wn_ref[...], preferred_element_type=_f32)
    sa = pre * _sigmoid(pre)
    sa_ref[...] = sa
    h0_ref[...] = jnp.dot(attrs_ref[...], we_ref[...],
                          preferred_element_type=_f32) + sa


def _build_tc_node0(interpret=False):
    return pl.pallas_call(
        _tc_node0_body,
        grid=(N // BN,),
        in_specs=[_nblk(DA), _nblk(1), _full((DA, HID)), _full((NEMB, HID))],
        out_specs=[_nblk(HID), _nblk(HID)],
        out_shape=[jax.ShapeDtypeStruct((N, HID), _f32),
                   jax.ShapeDtypeStruct((N, HID), _f32)],
        interpret=interpret,
    )


def _tc_rad_body(r2_ref, wr_ref, rad0_ref, rad1_ref):
    r = jnp.sqrt(r2_ref[...] + 1e-9)
    kr = r * _kvec()
    ef = jnp.sin(kr) / r
    rad0_ref[...] = jnp.dot(ef, wr_ref[0], preferred_element_type=_f32)
    rad1_ref[...] = jnp.dot(ef, wr_ref[1], preferred_element_type=_f32)


def _build_tc_rad(interpret=False):
    return pl.pallas_call(
        _tc_rad_body,
        grid=(E // BE,),
        in_specs=[_eblk(1), _full((2, NRAD, HID))],
        out_specs=[_eblk(HID), _eblk(HID)],
        out_shape=[jax.ShapeDtypeStruct((E, HID), _f32),
                   jax.ShapeDtypeStruct((E, HID), _f32)],
        interpret=interpret,
    )


def _tc_fwd_body(a0_ref, a1_ref, h_ref, sa_ref, wsc_ref, wlin_ref, wprod_ref,
                 wr_ref, t2_ref, hn_ref, ne_ref):
    h = h_ref[...]
    agg = a0_ref[...] + a1_ref[...]
    sc = jnp.dot(h, wsc_ref[...], preferred_element_type=_f32)
    t1 = jnp.dot(agg, wlin_ref[...], preferred_element_type=_f32)
    t2 = jnp.dot(t1, wprod_ref[...], preferred_element_type=_f32)
    nf = t2 * _sigmoid(t2) + sc
    t2_ref[...] = t2
    hn_ref[...] = nf + sa_ref[...]
    ne_ref[...] = jnp.dot(nf, wr_ref[...], preferred_element_type=_f32)


def _build_tc_fwd(interpret=False):
    agg_spec0 = pl.BlockSpec((BN, HID), lambda i: (i, 0))
    agg_spec1 = pl.BlockSpec((BN, HID), lambda i: (i + N // BN, 0))
    return pl.pallas_call(
        _tc_fwd_body,
        grid=(N // BN,),
        in_specs=[agg_spec0, agg_spec1, _nblk(HID), _nblk(HID),
                  _full((HID, HID)), _full((HID, HID)), _full((HID, HID)),
                  _full((HID, 1))],
        out_specs=[_nblk(HID), _nblk(HID), _nblk(1)],
        out_shape=[jax.ShapeDtypeStruct((N, HID), _f32),
                   jax.ShapeDtypeStruct((N, HID), _f32),
                   jax.ShapeDtypeStruct((N, 1), _f32)],
        interpret=interpret,
    )


def _tc_bwd1_body(t2_ref, wpt_ref, wlt_ref, wr_ref, gagg_ref):
    t2 = t2_ref[...]
    g_nf = jnp.broadcast_to(wr_ref[...], t2.shape)
    g_t2 = g_nf * _dsilu(t2)
    gagg_ref[...] = jnp.dot(jnp.dot(g_t2, wpt_ref[...],
                                    preferred_element_type=_f32),
                            wlt_ref[...], preferred_element_type=_f32)


def _build_tc_bwd1(interpret=False):
    return pl.pallas_call(
        _tc_bwd1_body,
        grid=(N // BN,),
        in_specs=[_nblk(HID), _full((HID, HID)), _full((HID, HID)),
                  _full((1, HID))],
        out_specs=[_nblk(HID)],
        out_shape=[jax.ShapeDtypeStruct((N, HID), _f32)],
        interpret=interpret,
    )


def _tc_bwd0_body(gm0_ref, gm1_ref, t2_ref, wr0_ref, ghsc1_ref,
                  wpt_ref, wlt_ref, wsct_ref, gagg_ref, ghsc_ref):
    t2 = t2_ref[...]
    g_in1 = gm0_ref[...] + gm1_ref[...] + ghsc1_ref[...]
    g_nf = g_in1 + wr0_ref[...]
    g_t2 = g_nf * _dsilu(t2)
    gagg_ref[...] = jnp.dot(jnp.dot(g_t2, wpt_ref[...],
                                    preferred_element_type=_f32),
                            wlt_ref[...], preferred_element_type=_f32)
    ghsc_ref[...] = jnp.dot(g_nf, wsct_ref[...], preferred_element_type=_f32)


def _build_tc_bwd0(interpret=False):
    gm_spec0 = pl.BlockSpec((BN, HID), lambda i: (i, 0))
    gm_spec1 = pl.BlockSpec((BN, HID), lambda i: (i + N // BN, 0))
    return pl.pallas_call(
        _tc_bwd0_body,
        grid=(N // BN,),
        in_specs=[gm_spec0, gm_spec1, _nblk(HID), _full((1, HID)),
                  _full((1, HID)), _full((HID, HID)), _full((HID, HID)),
                  _full((HID, HID))],
        out_specs=[_nblk(HID), _nblk(HID)],
        out_shape=[jax.ShapeDtypeStruct((N, HID), _f32),
                   jax.ShapeDtypeStruct((N, HID), _f32)],
        interpret=interpret,
    )


def _tc_outn_body(gm0_ref, gm1_ref, ghsc_ref, wet_ref, s_ref, attrs_ref,
                  ne0_ref, ne1_ref, nfo_ref, nao_ref, ne_ref):
    g_nf0 = gm0_ref[...] + gm1_ref[...] + ghsc_ref[...]
    ga = jnp.dot(g_nf0, wet_ref[...], preferred_element_type=_f32)
    nfo = -s_ref[...] * ga
    nfo_ref[...] = nfo
    nao_ref[...] = attrs_ref[...] + nfo
    ne_ref[...] = ne0_ref[...] + ne1_ref[...]


def _build_tc_outn(interpret=False):
    gm_spec0 = pl.BlockSpec((BN, HID), lambda i: (i, 0))
    gm_spec1 = pl.BlockSpec((BN, HID), lambda i: (i + N // BN, 0))
    return pl.pallas_call(
        _tc_outn_body,
        grid=(N // BN,),
        in_specs=[gm_spec0, gm_spec1, _nblk(HID), _full((HID, DA)),
                  _nblk(1), _nblk(DA), _nblk(1), _nblk(1)],
        out_specs=[_nblk(DA), _nblk(DA), _nblk(1)],
        out_shape=[jax.ShapeDtypeStruct((N, DA), _f32),
                   jax.ShapeDtypeStruct((N, DA), _f32),
                   jax.ShapeDtypeStruct((N, 1), _f32)],
        interpret=interpret,
    )


def _tc_gr_body(p0_ref, p1_ref, r2_ref, wr_ref, a_ref):
    r = jnp.sqrt(r2_ref[...] + 1e-9)
    kr = r * _kvec()
    defdr = (_kvec() * jnp.cos(kr)) / r - jnp.sin(kr) / (r * r)
    q0 = jnp.dot(defdr, wr_ref[0], preferred_element_type=_f32)
    q1 = jnp.dot(defdr, wr_ref[1], preferred_element_type=_f32)
    g_r = jnp.sum(p0_ref[...] * q0 + p1_ref[...] * q1, axis=1, keepdims=True)
    a_ref[...] = g_r / r


def _build_tc_gr(interpret=False):
    return pl.pallas_call(
        _tc_gr_body,
        grid=(E // BE,),
        in_specs=[_eblk(HID), _eblk(HID), _eblk(1), _full((2, NRAD, HID))],
        out_specs=[_eblk(1)],
        out_shape=[jax.ShapeDtypeStruct((E, 1), _f32)],
        interpret=interpret,
    )


def _tc_outp_body(gx0_ref, gx1_ref, gy0_ref, gy1_ref, gz0_ref, gz1_ref,
                  s_ref, pos_ref, f_ref, po_ref):
    sneg = -s_ref[...]
    fx = sneg * (gx0_ref[...] + gx1_ref[...])
    fy = sneg * (gy0_ref[...] + gy1_ref[...])
    fz = sneg * (gz0_ref[...] + gz1_ref[...])
    f = jnp.concatenate([fx, fy, fz], axis=1)
    f_ref[...] = f
    po_ref[...] = pos_ref[...] + f


def _build_tc_outp(interpret=False):
    pspec0 = pl.BlockSpec((BN, 1), lambda i: (i, 0))
    pspec1 = pl.BlockSpec((BN, 1), lambda i: (i + N // BN, 0))
    return pl.pallas_call(
        _tc_outp_body,
        grid=(N // BN,),
        in_specs=[pspec0, pspec1, pspec0, pspec1, pspec0, pspec1,
                  _nblk(1), _nblk(3)],
        out_specs=[_nblk(3), _nblk(3)],
        out_shape=[jax.ShapeDtypeStruct((N, 3), _f32),
                   jax.ShapeDtypeStruct((N, 3), _f32)],
        interpret=interpret,
    )


def _tc_seg_body(b_ref, ne0_ref, ne1_ref, con_ref, tot_ref):
    @pl.when(pl.program_id(0) == 0)
    def _init():
        con_ref[...] = jnp.zeros_like(con_ref)
        tot_ref[...] = jnp.zeros_like(tot_ref)

    oh = (b_ref[...] == _giota()).astype(_f32)
    dn = (((0,), (0,)), ((), ()))
    c0 = lax.dot_general(oh, ne0_ref[...], dn, preferred_element_type=_f32)
    c1 = lax.dot_general(oh, ne1_ref[...], dn, preferred_element_type=_f32)
    con_ref[...] += jnp.concatenate([c0, c1], axis=1)
    tot_ref[...] += c0 + c1


def _build_tc_seg(interpret=False):
    return pl.pallas_call(
        _tc_seg_body,
        grid=(N // BN,),
        in_specs=[_nblk(1), _nblk(1), _nblk(1)],
        out_specs=[pl.BlockSpec((G, 2), lambda i: (0, 0)),
                   pl.BlockSpec((G, 1), lambda i: (0, 0))],
        out_shape=[jax.ShapeDtypeStruct((G, 2), _f32),
                   jax.ShapeDtypeStruct((G, 1), _f32)],
        interpret=interpret,
    )


_tc_node0 = _build_tc_node0()
_tc_rad = _build_tc_rad()
_tc_fwd = _build_tc_fwd()
_tc_bwd1 = _build_tc_bwd1()
_tc_bwd0 = _build_tc_bwd0()
_tc_outn = _build_tc_outn()
_tc_gr = _build_tc_gr()
_tc_outp = _build_tc_outp()
_tc_seg = _build_tc_seg()


def kernel(positions, node_attrs, sigma, noise_pos, noise_attrs, edge_index,
           batch, ptr, W_embed, W_noise, W_rad, W_sc, W_lin, W_prod, w_read):
    s1 = sigma.reshape(N, 1)
    c_in = 1.0 - s1
    s2c = s1 * s1
    pos_in = c_in * positions + s2c * noise_pos
    attrs_in = c_in * node_attrs + s2c * noise_attrs
    pos4 = jnp.pad(pos_in, ((0, 0), (0, 1))).reshape(N * 4)
    src = edge_index[0].astype(_i32)
    dst = edge_index[1].astype(_i32)

    vx, vy, vz, r2 = _build_sc_pos_fwd()(pos4, src, dst)
    h0, sa = _tc_node0(attrs_in, s1, W_embed, W_noise)
    r2c = r2.reshape(E, 1)
    rad0, rad1 = _tc_rad(r2c, W_rad)
    aggp0 = _build_sc_msg_fwd()(h0, rad0, src, dst)
    t2_0, h1, ne0 = _tc_fwd(aggp0, aggp0, h0, sa, W_sc[0], W_lin[0],
                            W_prod[0], w_read[0].reshape(HID, 1))
    aggp1 = _build_sc_msg_fwd()(h1, rad1, src, dst)
    t2_1, _h2, ne1 = _tc_fwd(aggp1, aggp1, h1, sa, W_sc[1], W_lin[1],
                             W_prod[1], w_read[1].reshape(HID, 1))

    gagg1 = _tc_bwd1(t2_1, W_prod[1].T, W_lin[1].T, w_read[1].reshape(1, HID))
    ghm1, p1 = _build_sc_msg_bwd()(gagg1[0], h1, rad1, src, dst)
    ghsc1_row = (w_read[1] @ W_sc[1].T).reshape(1, HID)
    gagg0, ghsc0 = _tc_bwd0(ghm1, ghm1, t2_0, w_read[0].reshape(1, HID),
                            ghsc1_row, W_prod[0].T, W_lin[0].T, W_sc[0].T)
    ghm0, p0 = _build_sc_msg_bwd()(gagg0, h0, rad0, src, dst)
    a_coef = _tc_gr(p0, p1, r2c, W_rad)[0]
    gpx, gpy, gpz = _build_sc_pos_bwd()(vx, vy, vz, a_coef.reshape(E), src,
                                        dst)

    node_forces, node_attrs_out, node_e = _tc_outn(
        ghm0, ghm0, ghsc0, W_embed.T, s1, attrs_in,
        ne0, ne1)
    gpx = gpx.reshape(NC * N, 1)
    gpy = gpy.reshape(NC * N, 1)
    gpz = gpz.reshape(NC * N, 1)
    forces, pos_out = _tc_outp(gpx, gpx, gpy, gpy, gpz, gpz, s1, pos_in)
    contrib, total = _tc_seg(batch.reshape(N, 1).astype(_i32), ne0, ne1)

    return (total.reshape(G), node_e.reshape(N), contrib,
            forces, node_forces, pos_out, node_attrs_out)
